# baseline (speedup 1.0000x reference)
import functools

import jax
import jax.numpy as jnp
from jax import lax
from jax.experimental import pallas as pl
from jax.experimental.pallas import tpu as pltpu
from jax.experimental.pallas import tpu_sc as plsc

_EPS = 1e-12


def _sc_gather_pack(table, ids3, nw, nch, ch, h):
    hp = ch // 2
    nk = h // 16
    mesh = plsc.VectorSubcoreMesh(core_axis_name="c", subcore_axis_name="s")
    info = plsc.get_sparse_core_info()
    nc = info.num_cores

    @functools.partial(
        pl.kernel,
        mesh=mesh,
        out_type=jax.ShapeDtypeStruct((nw * nch * hp, h), jnp.uint32),
        compiler_params=pltpu.CompilerParams(needs_layout_passes=False),
        scratch_types=[
            pltpu.VMEM((nch, ch), jnp.int32),
            pltpu.VMEM((ch, h), jnp.float32),
            pltpu.VMEM((ch, h), jnp.float32),
            pltpu.VMEM((hp, h), jnp.uint32),
            pltpu.SemaphoreType.DMA,
            pltpu.SemaphoreType.DMA,
            pltpu.SemaphoreType.DMA,
        ],
    )
    def gather_kernel(table_hbm, ids_hbm, out_hbm, idx_v, rows0, rows1, pk, semg0, semg1, semo):
        wid = lax.axis_index("s") * nc + lax.axis_index("c")
        pltpu.sync_copy(ids_hbm.at[wid], idx_v)
        rbufs = (rows0, rows1)
        gsems = (semg0, semg1)
        ocp = None
        cp = pltpu.async_copy(table_hbm.at[idx_v.at[0]], rbufs[0], gsems[0])
        for c in range(nch):
            cp.wait()
            if c + 1 < nch:
                cp = pltpu.async_copy(
                    table_hbm.at[idx_v.at[c + 1]], rbufs[(c + 1) % 2], gsems[(c + 1) % 2]
                )
            if ocp is not None:
                ocp.wait()
            rows = rbufs[c % 2]

            @plsc.parallel_loop(0, hp, unroll=2)
            def _(r):
                half = jnp.uint32(0x8000)
                mask = jnp.uint32(0xFFFF0000)
                for k in range(nk):
                    a = plsc.bitcast(rows[r, pl.ds(16 * k, 16)], jnp.uint32)
                    b = plsc.bitcast(rows[r + hp, pl.ds(16 * k, 16)], jnp.uint32)
                    pk[r, pl.ds(16 * k, 16)] = ((a + half) >> 16) | ((b + half) & mask)
            ocp = pltpu.async_copy(
                pk, out_hbm.at[pl.ds(wid * nch * hp + c * hp, hp)], semo
            )
        ocp.wait()

    return gather_kernel(table, ids3)


def _col_sumsq(table, v, h, vb):
    g = v // vb

    def body(x_ref, o_ref):
        i = pl.program_id(0)

        @pl.when(i == 0)
        def _():
            o_ref[...] = jnp.zeros_like(o_ref)

        x = x_ref[...]
        o_ref[...] += jnp.sum(x * x, axis=0, keepdims=True)

    return pl.pallas_call(
        body,
        grid=(g,),
        in_specs=[pl.BlockSpec((vb, h), lambda i: (i, 0))],
        out_specs=pl.BlockSpec((1, h), lambda i: (0, 0)),
        out_shape=jax.ShapeDtypeStruct((1, h), jnp.float32),
    )(table)


def _pointwise(packed, norm2, pos_table, tt_row, gamma, beta, bias, n, s, h, tb, ch):
    g = n // tb
    pb = s // tb
    hp = ch // 2
    ng = tb // ch

    def body(r_ref, n2_ref, p_ref, tt_ref, g_ref, b_ref, bias_ref, o_ref):
        pk = r_ref[...]
        lo = lax.bitcast_convert_type(pk << 16, jnp.float32)
        hi = lax.bitcast_convert_type(pk & jnp.uint32(0xFFFF0000), jnp.float32)
        pieces = []
        for gi in range(ng):
            pieces.append(lo[gi * hp : (gi + 1) * hp])
            pieces.append(hi[gi * hp : (gi + 1) * hp])
        x = jnp.concatenate(pieces, axis=0)
        nn = jnp.sqrt(n2_ref[...])
        x = x - bias_ref[...] - nn
        inv = lax.rsqrt(jnp.sum(x * x, axis=-1, keepdims=True))
        x = x * inv + p_ref[...] + tt_ref[...]
        m = jnp.mean(x, axis=-1, keepdims=True)
        xc = x - m
        var = jnp.mean(xc * xc, axis=-1, keepdims=True)
        o_ref[...] = xc * lax.rsqrt(var + _EPS) * g_ref[...] + b_ref[...]

    return pl.pallas_call(
        body,
        grid=(g,),
        in_specs=[
            pl.BlockSpec((tb // 2, h), lambda i: (i, 0)),
            pl.BlockSpec((1, h), lambda i: (0, 0)),
            pl.BlockSpec((tb, h), lambda i: (i % pb, 0)),
            pl.BlockSpec((1, h), lambda i: (0, 0)),
            pl.BlockSpec((1, h), lambda i: (0, 0)),
            pl.BlockSpec((1, h), lambda i: (0, 0)),
            pl.BlockSpec((1, h), lambda i: (0, 0)),
        ],
        out_specs=pl.BlockSpec((tb, h), lambda i: (i, 0)),
        out_shape=jax.ShapeDtypeStruct((n, h), jnp.float32),
    )(packed, norm2, pos_table, tt_row, gamma, beta, bias)


def kernel(input_ids, word_table, pos_table, tt_table, ln_gamma, ln_beta, bias_subspace):
    b, s = input_ids.shape
    v, h = word_table.shape
    n = b * s

    nw = 32
    ch = 64
    nch = n // (nw * ch)
    ids3 = input_ids.reshape(nw, nch, ch).astype(jnp.int32)

    packed = _sc_gather_pack(word_table, ids3, nw, nch, ch, h)
    norm2 = _col_sumsq(word_table, v, h, vb=4000)
    out = _pointwise(
        packed,
        norm2,
        pos_table,
        tt_table[0:1],
        ln_gamma.reshape(1, h),
        ln_beta.reshape(1, h),
        bias_subspace.reshape(1, h),
        n,
        s,
        h,
        tb=256,
        ch=ch,
    )
    return out.reshape(b, s, h)


# speedup vs baseline: 1.0020x; 1.0020x over previous
import functools

import jax
import jax.numpy as jnp
from jax import lax
from jax.experimental import pallas as pl
from jax.experimental.pallas import tpu as pltpu
from jax.experimental.pallas import tpu_sc as plsc

_EPS = 1e-12


def _sc_gather_pack(table, ids3, nw, nch, ch, h):
    hp = ch // 2
    nk = h // 16
    mesh = plsc.VectorSubcoreMesh(core_axis_name="c", subcore_axis_name="s")
    info = plsc.get_sparse_core_info()
    nc = info.num_cores

    @functools.partial(
        pl.kernel,
        mesh=mesh,
        out_type=jax.ShapeDtypeStruct((nw * nch * hp, h), jnp.uint32),
        compiler_params=pltpu.CompilerParams(needs_layout_passes=False),
        scratch_types=[
            pltpu.VMEM((nch, ch), jnp.int32),
            pltpu.VMEM((ch, h), jnp.float32),
            pltpu.VMEM((ch, h), jnp.float32),
            pltpu.VMEM((hp, h), jnp.uint32),
            pltpu.SemaphoreType.DMA,
            pltpu.SemaphoreType.DMA,
            pltpu.SemaphoreType.DMA,
        ],
    )
    def gather_kernel(table_hbm, ids_hbm, out_hbm, idx_v, rows0, rows1, pk, semg0, semg1, semo):
        wid = lax.axis_index("s") * nc + lax.axis_index("c")
        pltpu.sync_copy(ids_hbm.at[wid], idx_v)
        rbufs = (rows0, rows1)
        gsems = (semg0, semg1)
        ocp = None
        cp = pltpu.async_copy(table_hbm.at[idx_v.at[0]], rbufs[0], gsems[0])
        for c in range(nch):
            cp.wait()
            if c + 1 < nch:
                cp = pltpu.async_copy(
                    table_hbm.at[idx_v.at[c + 1]], rbufs[(c + 1) % 2], gsems[(c + 1) % 2]
                )
            if ocp is not None:
                ocp.wait()
            rows = rbufs[c % 2]

            @plsc.parallel_loop(0, hp, unroll=4)
            def _(r):
                mask = jnp.uint32(0xFFFF0000)
                for k in range(nk):
                    a = plsc.bitcast(rows[r, pl.ds(16 * k, 16)], jnp.uint32)
                    b = plsc.bitcast(rows[r + hp, pl.ds(16 * k, 16)], jnp.uint32)
                    pk[r, pl.ds(16 * k, 16)] = (a >> 16) | (b & mask)
            ocp = pltpu.async_copy(
                pk, out_hbm.at[pl.ds(wid * nch * hp + c * hp, hp)], semo
            )
        ocp.wait()

    return gather_kernel(table, ids3)


def _col_sumsq(table, v, h, vb):
    g = v // vb

    def body(x_ref, o_ref):
        i = pl.program_id(0)

        @pl.when(i == 0)
        def _():
            o_ref[...] = jnp.zeros_like(o_ref)

        x = x_ref[...]
        o_ref[...] += jnp.sum(x * x, axis=0, keepdims=True)

    return pl.pallas_call(
        body,
        grid=(g,),
        in_specs=[pl.BlockSpec((vb, h), lambda i: (i, 0))],
        out_specs=pl.BlockSpec((1, h), lambda i: (0, 0)),
        out_shape=jax.ShapeDtypeStruct((1, h), jnp.float32),
    )(table)


def _pointwise(packed, norm2, pos_table, tt_row, gamma, beta, bias, n, s, h, tb, ch):
    g = n // tb
    pb = s // tb
    hp = ch // 2
    ng = tb // ch

    def body(r_ref, n2_ref, p_ref, tt_ref, g_ref, b_ref, bias_ref, o_ref):
        pk = r_ref[...]
        lo = lax.bitcast_convert_type(pk << 16, jnp.float32)
        hi = lax.bitcast_convert_type(pk & jnp.uint32(0xFFFF0000), jnp.float32)
        pieces = []
        for gi in range(ng):
            pieces.append(lo[gi * hp : (gi + 1) * hp])
            pieces.append(hi[gi * hp : (gi + 1) * hp])
        x = jnp.concatenate(pieces, axis=0)
        nn = jnp.sqrt(n2_ref[...])
        x = x - bias_ref[...] - nn
        inv = lax.rsqrt(jnp.sum(x * x, axis=-1, keepdims=True))
        x = x * inv + p_ref[...] + tt_ref[...]
        m = jnp.mean(x, axis=-1, keepdims=True)
        xc = x - m
        var = jnp.mean(xc * xc, axis=-1, keepdims=True)
        o_ref[...] = xc * lax.rsqrt(var + _EPS) * g_ref[...] + b_ref[...]

    return pl.pallas_call(
        body,
        grid=(g,),
        in_specs=[
            pl.BlockSpec((tb // 2, h), lambda i: (i, 0)),
            pl.BlockSpec((1, h), lambda i: (0, 0)),
            pl.BlockSpec((tb, h), lambda i: (i % pb, 0)),
            pl.BlockSpec((1, h), lambda i: (0, 0)),
            pl.BlockSpec((1, h), lambda i: (0, 0)),
            pl.BlockSpec((1, h), lambda i: (0, 0)),
            pl.BlockSpec((1, h), lambda i: (0, 0)),
        ],
        out_specs=pl.BlockSpec((tb, h), lambda i: (i, 0)),
        out_shape=jax.ShapeDtypeStruct((n, h), jnp.float32),
    )(packed, norm2, pos_table, tt_row, gamma, beta, bias)


def kernel(input_ids, word_table, pos_table, tt_table, ln_gamma, ln_beta, bias_subspace):
    b, s = input_ids.shape
    v, h = word_table.shape
    n = b * s

    nw = 32
    ch = 64
    nch = n // (nw * ch)
    ids3 = input_ids.reshape(nw, nch, ch).astype(jnp.int32)

    packed = _sc_gather_pack(word_table, ids3, nw, nch, ch, h)
    norm2 = _col_sumsq(word_table, v, h, vb=4000)
    out = _pointwise(
        packed,
        norm2,
        pos_table,
        tt_table[0:1],
        ln_gamma.reshape(1, h),
        ln_beta.reshape(1, h),
        bias_subspace.reshape(1, h),
        n,
        s,
        h,
        tb=256,
        ch=ch,
    )
    return out.reshape(b, s, h)


# speedup vs baseline: 1.0279x; 1.0259x over previous
import functools

import jax
import jax.numpy as jnp
from jax import lax
from jax.experimental import pallas as pl
from jax.experimental.pallas import tpu as pltpu
from jax.experimental.pallas import tpu_sc as plsc

_EPS = 1e-12


def _sc_gather_pack(table, ids3, nw, nch, ch, h):
    hp = ch // 2
    nk = h // 16
    mesh = plsc.VectorSubcoreMesh(core_axis_name="c", subcore_axis_name="s")
    info = plsc.get_sparse_core_info()
    nc = info.num_cores

    @functools.partial(
        pl.kernel,
        mesh=mesh,
        out_type=jax.ShapeDtypeStruct((nw * nch * hp, h), jnp.uint32),
        compiler_params=pltpu.CompilerParams(needs_layout_passes=False),
        scratch_types=[
            pltpu.VMEM((nch, ch), jnp.int32),
            pltpu.VMEM((ch, h), jnp.float32),
            pltpu.VMEM((ch, h), jnp.float32),
            pltpu.VMEM((hp, h), jnp.uint32),
            pltpu.SemaphoreType.DMA,
            pltpu.SemaphoreType.DMA,
            pltpu.SemaphoreType.DMA,
        ],
    )
    def gather_kernel(table_hbm, ids_hbm, out_hbm, idx_v, rows0, rows1, pk, semg0, semg1, semo):
        wid = lax.axis_index("s") * nc + lax.axis_index("c")
        pltpu.sync_copy(ids_hbm.at[wid], idx_v)
        rbufs = (rows0, rows1)
        gsems = (semg0, semg1)
        ocp = None
        cp = pltpu.async_copy(table_hbm.at[idx_v.at[0]], rbufs[0], gsems[0])
        for c in range(nch):
            cp.wait()
            if c + 1 < nch:
                cp = pltpu.async_copy(
                    table_hbm.at[idx_v.at[c + 1]], rbufs[(c + 1) % 2], gsems[(c + 1) % 2]
                )
            if ocp is not None:
                ocp.wait()
            rows = rbufs[c % 2]

            @plsc.parallel_loop(0, hp, unroll=4)
            def _(r):
                mask = jnp.uint32(0xFFFF0000)
                for k in range(nk):
                    a = plsc.bitcast(rows[r, pl.ds(16 * k, 16)], jnp.uint32)
                    b = plsc.bitcast(rows[r + hp, pl.ds(16 * k, 16)], jnp.uint32)
                    half = jnp.uint32(0x8000)
                    pk[r, pl.ds(16 * k, 16)] = ((a + half) >> 16) | ((b + half) & mask)
            ocp = pltpu.async_copy(
                pk, out_hbm.at[pl.ds(wid * nch * hp + c * hp, hp)], semo
            )
        ocp.wait()

    return gather_kernel(table, ids3)


def _col_sumsq(table, v, h, vb):
    g = v // vb

    def body(x_ref, o_ref):
        i = pl.program_id(0)

        @pl.when(i == 0)
        def _():
            o_ref[...] = jnp.zeros_like(o_ref)

        x = x_ref[...]
        o_ref[...] += jnp.sum(x * x, axis=0, keepdims=True)

    return pl.pallas_call(
        body,
        grid=(g,),
        in_specs=[pl.BlockSpec((vb, h), lambda i: (i, 0))],
        out_specs=pl.BlockSpec((1, h), lambda i: (0, 0)),
        out_shape=jax.ShapeDtypeStruct((1, h), jnp.float32),
    )(table)


def _pointwise(packed, norm2, pos_table, tt_row, gamma, beta, bias, n, s, h, tb, ch):
    g = n // tb
    pb = s // tb
    nb = g // pb
    hp = ch // 2
    ng = tb // ch

    def tok_idx(i):
        return (i % nb) * pb + i // nb

    def body(r_ref, n2_ref, p_ref, tt_ref, g_ref, b_ref, bias_ref, o_ref):
        pk = r_ref[...]
        lo = lax.bitcast_convert_type(pk << 16, jnp.float32)
        hi = lax.bitcast_convert_type(pk & jnp.uint32(0xFFFF0000), jnp.float32)
        pieces = []
        for gi in range(ng):
            pieces.append(lo[gi * hp : (gi + 1) * hp])
            pieces.append(hi[gi * hp : (gi + 1) * hp])
        x = jnp.concatenate(pieces, axis=0)
        nn = jnp.sqrt(n2_ref[...])
        x = x - bias_ref[...] - nn
        inv = lax.rsqrt(jnp.sum(x * x, axis=-1, keepdims=True))
        x = x * inv + p_ref[...] + tt_ref[...]
        m = jnp.mean(x, axis=-1, keepdims=True)
        xc = x - m
        var = jnp.mean(xc * xc, axis=-1, keepdims=True)
        o_ref[...] = xc * lax.rsqrt(var + _EPS) * g_ref[...] + b_ref[...]

    return pl.pallas_call(
        body,
        grid=(g,),
        in_specs=[
            pl.BlockSpec((tb // 2, h), lambda i: (tok_idx(i), 0)),
            pl.BlockSpec((1, h), lambda i: (0, 0)),
            pl.BlockSpec((tb, h), lambda i: (i // nb, 0)),
            pl.BlockSpec((1, h), lambda i: (0, 0)),
            pl.BlockSpec((1, h), lambda i: (0, 0)),
            pl.BlockSpec((1, h), lambda i: (0, 0)),
            pl.BlockSpec((1, h), lambda i: (0, 0)),
        ],
        out_specs=pl.BlockSpec((tb, h), lambda i: (tok_idx(i), 0)),
        out_shape=jax.ShapeDtypeStruct((n, h), jnp.float32),
    )(packed, norm2, pos_table, tt_row, gamma, beta, bias)


def kernel(input_ids, word_table, pos_table, tt_table, ln_gamma, ln_beta, bias_subspace):
    b, s = input_ids.shape
    v, h = word_table.shape
    n = b * s

    nw = 32
    ch = 64
    nch = n // (nw * ch)
    ids3 = input_ids.reshape(nw, nch, ch).astype(jnp.int32)

    packed = _sc_gather_pack(word_table, ids3, nw, nch, ch, h)
    norm2 = _col_sumsq(word_table, v, h, vb=4000)
    out = _pointwise(
        packed,
        norm2,
        pos_table,
        tt_table[0:1],
        ln_gamma.reshape(1, h),
        ln_beta.reshape(1, h),
        bias_subspace.reshape(1, h),
        n,
        s,
        h,
        tb=256,
        ch=ch,
    )
    return out.reshape(b, s, h)


# speedup vs baseline: 1.0802x; 1.0508x over previous
import functools

import jax
import jax.numpy as jnp
from jax import lax
from jax.experimental import pallas as pl
from jax.experimental.pallas import tpu as pltpu
from jax.experimental.pallas import tpu_sc as plsc

_EPS = 1e-12


def _sc_gather_pack(table, ids3, nw, nch, ch, h):
    hp = ch // 2
    nk = h // 16
    mesh = plsc.VectorSubcoreMesh(core_axis_name="c", subcore_axis_name="s")
    info = plsc.get_sparse_core_info()
    nc = info.num_cores

    @functools.partial(
        pl.kernel,
        mesh=mesh,
        out_type=jax.ShapeDtypeStruct((nw * nch * hp, h), jnp.uint32),
        compiler_params=pltpu.CompilerParams(needs_layout_passes=False),
        scratch_types=[
            pltpu.VMEM((nch, ch), jnp.int32),
            pltpu.VMEM((ch, h), jnp.float32),
            pltpu.VMEM((ch, h), jnp.float32),
            pltpu.VMEM((hp, h), jnp.uint32),
            pltpu.SemaphoreType.DMA,
            pltpu.SemaphoreType.DMA,
            pltpu.SemaphoreType.DMA,
        ],
    )
    def gather_kernel(table_hbm, ids_hbm, out_hbm, idx_v, rows0, rows1, pk, semg0, semg1, semo):
        wid = lax.axis_index("s") * nc + lax.axis_index("c")
        pltpu.sync_copy(ids_hbm.at[wid], idx_v)
        rbufs = (rows0, rows1)
        gsems = (semg0, semg1)
        ocp = None
        cp = pltpu.async_copy(table_hbm.at[idx_v.at[0]], rbufs[0], gsems[0])
        for c in range(nch):
            cp.wait()
            if c + 1 < nch:
                cp = pltpu.async_copy(
                    table_hbm.at[idx_v.at[c + 1]], rbufs[(c + 1) % 2], gsems[(c + 1) % 2]
                )
            if ocp is not None:
                ocp.wait()
            rows = rbufs[c % 2]

            @plsc.parallel_loop(0, hp, unroll=4)
            def _(r):
                mask = jnp.uint32(0xFFFF0000)
                for k in range(nk):
                    a = plsc.bitcast(rows[r, pl.ds(16 * k, 16)], jnp.uint32)
                    b = plsc.bitcast(rows[r + hp, pl.ds(16 * k, 16)], jnp.uint32)
                    half = jnp.uint32(0x8000)
                    pk[r, pl.ds(16 * k, 16)] = ((a + half) >> 16) | ((b + half) & mask)
            ocp = pltpu.async_copy(
                pk, out_hbm.at[pl.ds(wid * nch * hp + c * hp, hp)], semo
            )
        ocp.wait()

    return gather_kernel(table, ids3)


def _col_sumsq(table, v, h, vb):
    g = v // vb

    def body(x_ref, o_ref):
        i = pl.program_id(0)

        @pl.when(i == 0)
        def _():
            o_ref[...] = jnp.zeros_like(o_ref)

        x = x_ref[...]
        o_ref[...] += jnp.sum(x * x, axis=0, keepdims=True)

    return pl.pallas_call(
        body,
        grid=(g,),
        in_specs=[pl.BlockSpec((vb, h), lambda i: (i, 0))],
        out_specs=pl.BlockSpec((1, h), lambda i: (0, 0)),
        out_shape=jax.ShapeDtypeStruct((1, h), jnp.float32),
    )(table)


def _pointwise(packed, norm2, pos_table, tt_row, gamma, beta, bias, n, s, h, tb, ch):
    g = n // tb
    pb = s // tb
    nb = g // pb
    hp = ch // 2
    ng = tb // ch

    def tok_idx(i):
        return (i % nb) * pb + i // nb

    def body(r_ref, n2_ref, p_ref, tt_ref, g_ref, b_ref, bias_ref, o_ref):
        pk = r_ref[...]
        lo = lax.bitcast_convert_type(pk << 16, jnp.float32)
        hi = lax.bitcast_convert_type(pk & jnp.uint32(0xFFFF0000), jnp.float32)
        pieces = []
        for gi in range(ng):
            pieces.append(lo[gi * hp : (gi + 1) * hp])
            pieces.append(hi[gi * hp : (gi + 1) * hp])
        x = jnp.concatenate(pieces, axis=0)
        nn = jnp.sqrt(n2_ref[...])
        x = x - bias_ref[...] - nn
        inv = lax.rsqrt(jnp.sum(x * x, axis=-1, keepdims=True))
        x = x * inv + p_ref[...] + tt_ref[...]
        m = jnp.mean(x, axis=-1, keepdims=True)
        xc = x - m
        var = jnp.mean(xc * xc, axis=-1, keepdims=True)
        o_ref[...] = xc * lax.rsqrt(var + _EPS) * g_ref[...] + b_ref[...]

    return pl.pallas_call(
        body,
        grid=(g,),
        in_specs=[
            pl.BlockSpec((tb // 2, h), lambda i: (tok_idx(i), 0)),
            pl.BlockSpec((1, h), lambda i: (0, 0)),
            pl.BlockSpec((tb, h), lambda i: (i // nb, 0)),
            pl.BlockSpec((1, h), lambda i: (0, 0)),
            pl.BlockSpec((1, h), lambda i: (0, 0)),
            pl.BlockSpec((1, h), lambda i: (0, 0)),
            pl.BlockSpec((1, h), lambda i: (0, 0)),
        ],
        out_specs=pl.BlockSpec((tb, h), lambda i: (tok_idx(i), 0)),
        out_shape=jax.ShapeDtypeStruct((n, h), jnp.float32),
    )(packed, norm2, pos_table, tt_row, gamma, beta, bias)


def kernel(input_ids, word_table, pos_table, tt_table, ln_gamma, ln_beta, bias_subspace):
    b, s = input_ids.shape
    v, h = word_table.shape
    n = b * s

    nw = 32
    ch = 64
    nch = n // (nw * ch)
    ids3 = input_ids.reshape(nw, nch, ch).astype(jnp.int32)

    packed = _sc_gather_pack(word_table, ids3, nw, nch, ch, h)
    norm2 = _col_sumsq(word_table, v, h, vb=4000)
    out = _pointwise(
        packed,
        norm2,
        pos_table,
        tt_table[0:1],
        ln_gamma.reshape(1, h),
        ln_beta.reshape(1, h),
        bias_subspace.reshape(1, h),
        n,
        s,
        h,
        tb=512,
        ch=ch,
    )
    return out.reshape(b, s, h)


# speedup vs baseline: 1.1192x; 1.0361x over previous
import functools

import jax
import jax.numpy as jnp
from jax import lax
from jax.experimental import pallas as pl
from jax.experimental.pallas import tpu as pltpu
from jax.experimental.pallas import tpu_sc as plsc

_EPS = 1e-12


def _sc_gather_pack(table, ids3, nw, nch, ch, h):
    hp = ch // 2
    nk = h // 16
    mesh = plsc.VectorSubcoreMesh(core_axis_name="c", subcore_axis_name="s")
    info = plsc.get_sparse_core_info()
    nc = info.num_cores

    @functools.partial(
        pl.kernel,
        mesh=mesh,
        out_type=jax.ShapeDtypeStruct((nw * nch * hp, h), jnp.uint32),
        compiler_params=pltpu.CompilerParams(needs_layout_passes=False),
        scratch_types=[
            pltpu.VMEM((nch, ch), jnp.int32),
            pltpu.VMEM((ch, h), jnp.float32),
            pltpu.VMEM((ch, h), jnp.float32),
            pltpu.VMEM((hp, h), jnp.uint32),
            pltpu.SemaphoreType.DMA,
            pltpu.SemaphoreType.DMA,
            pltpu.SemaphoreType.DMA,
        ],
    )
    def gather_kernel(table_hbm, ids_hbm, out_hbm, idx_v, rows0, rows1, pk, semg0, semg1, semo):
        wid = lax.axis_index("s") * nc + lax.axis_index("c")
        pltpu.sync_copy(ids_hbm.at[wid], idx_v)
        rbufs = (rows0, rows1)
        gsems = (semg0, semg1)
        ocp = None
        cp = pltpu.async_copy(table_hbm.at[idx_v.at[0]], rbufs[0], gsems[0])
        for c in range(nch):
            cp.wait()
            if c + 1 < nch:
                cp = pltpu.async_copy(
                    table_hbm.at[idx_v.at[c + 1]], rbufs[(c + 1) % 2], gsems[(c + 1) % 2]
                )
            if ocp is not None:
                ocp.wait()
            rows = rbufs[c % 2]

            @plsc.parallel_loop(0, hp, unroll=4)
            def _(r):
                mask = jnp.uint32(0xFFFF0000)
                for k in range(nk):
                    a = plsc.bitcast(rows[r, pl.ds(16 * k, 16)], jnp.uint32)
                    b = plsc.bitcast(rows[r + hp, pl.ds(16 * k, 16)], jnp.uint32)
                    half = jnp.uint32(0x8000)
                    pk[r, pl.ds(16 * k, 16)] = ((a + half) >> 16) | ((b + half) & mask)
            ocp = pltpu.async_copy(
                pk, out_hbm.at[pl.ds(wid * nch * hp + c * hp, hp)], semo
            )
        ocp.wait()

    return gather_kernel(table, ids3)


def _col_sumsq(table, v, h, vb):
    g = v // vb

    def body(x_ref, o_ref):
        i = pl.program_id(0)

        @pl.when(i == 0)
        def _():
            o_ref[...] = jnp.zeros_like(o_ref)

        x = x_ref[...]
        o_ref[...] += jnp.sum(x * x, axis=0, keepdims=True)

    return pl.pallas_call(
        body,
        grid=(g,),
        in_specs=[pl.BlockSpec((vb, h), lambda i: (i, 0))],
        out_specs=pl.BlockSpec((1, h), lambda i: (0, 0)),
        out_shape=jax.ShapeDtypeStruct((1, h), jnp.float32),
    )(table)


def _pointwise(packed, norm2, pos_table, tt_row, gamma, beta, bias, n, s, h, tb, ch):
    g = n // tb
    pb = s // tb
    nb = g // pb
    hp = ch // 2
    ng = tb // ch

    def tok_idx(i):
        return (i % nb) * pb + i // nb

    def body(r_ref, n2_ref, p_ref, tt_ref, g_ref, b_ref, bias_ref, o_ref):
        pk = r_ref[...]
        lo = lax.bitcast_convert_type(pk << 16, jnp.float32)
        hi = lax.bitcast_convert_type(pk & jnp.uint32(0xFFFF0000), jnp.float32)
        pieces = []
        for gi in range(ng):
            pieces.append(lo[gi * hp : (gi + 1) * hp])
            pieces.append(hi[gi * hp : (gi + 1) * hp])
        x = jnp.concatenate(pieces, axis=0)
        nn = jnp.sqrt(n2_ref[...])
        x = x - bias_ref[...] - nn
        inv = lax.rsqrt(jnp.sum(x * x, axis=-1, keepdims=True))
        x = x * inv + p_ref[...] + tt_ref[...]
        m = jnp.mean(x, axis=-1, keepdims=True)
        xc = x - m
        var = jnp.mean(xc * xc, axis=-1, keepdims=True)
        o_ref[...] = xc * lax.rsqrt(var + _EPS) * g_ref[...] + b_ref[...]

    return pl.pallas_call(
        body,
        grid=(g,),
        in_specs=[
            pl.BlockSpec((tb // 2, h), lambda i: (tok_idx(i), 0)),
            pl.BlockSpec((1, h), lambda i: (0, 0)),
            pl.BlockSpec((tb, h), lambda i: (i // nb, 0)),
            pl.BlockSpec((1, h), lambda i: (0, 0)),
            pl.BlockSpec((1, h), lambda i: (0, 0)),
            pl.BlockSpec((1, h), lambda i: (0, 0)),
            pl.BlockSpec((1, h), lambda i: (0, 0)),
        ],
        out_specs=pl.BlockSpec((tb, h), lambda i: (tok_idx(i), 0)),
        out_shape=jax.ShapeDtypeStruct((n, h), jnp.float32),
    )(packed, norm2, pos_table, tt_row, gamma, beta, bias)


def kernel(input_ids, word_table, pos_table, tt_table, ln_gamma, ln_beta, bias_subspace):
    b, s = input_ids.shape
    v, h = word_table.shape
    n = b * s

    nw = 32
    ch = 64
    nch = n // (nw * ch)
    ids3 = input_ids.reshape(nw, nch, ch).astype(jnp.int32)

    packed = _sc_gather_pack(word_table, ids3, nw, nch, ch, h)
    norm2 = _col_sumsq(word_table, v, h, vb=4000)
    out = _pointwise(
        packed,
        norm2,
        pos_table,
        tt_table[0:1],
        ln_gamma.reshape(1, h),
        ln_beta.reshape(1, h),
        bias_subspace.reshape(1, h),
        n,
        s,
        h,
        tb=1024,
        ch=ch,
    )
    return out.reshape(b, s, h)
